# baseline (speedup 1.0000x reference)
import functools

import jax
import jax.numpy as jnp
from jax import lax
from jax.experimental import pallas as pl
from jax.experimental.pallas import tpu as pltpu
from jax.experimental.pallas import tpu_sc as plsc

B = 16384
U = 1000000
DG = 16
DM = 64

_info = plsc.get_sparse_core_info()
NC = _info.num_cores
NS = _info.num_subcores
NW = NC * NS
RPW = B // NW
CH = 128
NCH = RPW // CH

UB = 8192
NUB = -(-U // UB)


def _pair_pack(x_ref):
    xb = x_ref[...].astype(jnp.bfloat16).T
    return pltpu.bitcast(xb, jnp.float32)


def _pk_body(tm_ref, tg_ref, o_ref):
    pad = jnp.zeros((UB // 2, 128 - DM - DG), jnp.float32)
    o_ref[...] = jnp.concatenate(
        [_pair_pack(tm_ref), _pair_pack(tg_ref), pad], axis=1)


def _tc_pack(tmT, tgT):
    col = lambda i: (0, i)
    row = lambda i: (i, 0)
    return pl.pallas_call(
        _pk_body,
        grid=(NUB,),
        in_specs=[
            pl.BlockSpec((DM, UB), col),
            pl.BlockSpec((DG, UB), col),
        ],
        out_specs=pl.BlockSpec((UB // 2, 128), row),
        out_shape=jax.ShapeDtypeStruct((U // 2, 128), jnp.float32),
    )(tmT, tgT)


@functools.partial(
    pl.kernel,
    out_type=jax.ShapeDtypeStruct((B, 128), jnp.float32),
    mesh=plsc.VectorSubcoreMesh(core_axis_name="c", subcore_axis_name="s"),
    scratch_types=[
        pltpu.VMEM((NCH, CH), jnp.int32),
        pltpu.VMEM((CH, 128), jnp.float32),
        pltpu.VMEM((CH, 128), jnp.float32),
        pltpu.SemaphoreType.DMA,
        pltpu.SemaphoreType.DMA,
    ],
)
def _sc_gather(idx_hbm, tab_hbm, rows_out, idx, b0, b1, s0, s1):
    wid = lax.axis_index("s") * NC + lax.axis_index("c")
    base = wid * RPW
    pltpu.sync_copy(idx_hbm.at[wid], idx)
    bufs, sems = (b0, b1), (s0, s1)
    cps = [None, None]
    for j in range(NCH):
        k = j & 1
        cps[k] = pltpu.async_copy(tab_hbm.at[idx.at[j]], bufs[k], sems[k])
        if j > 0:
            pk = 1 - k
            cps[pk].wait()
            pltpu.sync_copy(
                bufs[pk], rows_out.at[pl.ds(base + (j - 1) * CH, CH)])
    cps[(NCH - 1) & 1].wait()
    pltpu.sync_copy(
        bufs[(NCH - 1) & 1], rows_out.at[pl.ds(base + (NCH - 1) * CH, CH)])


def _unpack_half(x, sel_hi):
    xi = jax.lax.bitcast_convert_type(x, jnp.int32)
    hi = jax.lax.bitcast_convert_type((xi >> 16) << 16, jnp.float32)
    lo = jax.lax.bitcast_convert_type(xi << 16, jnp.float32)
    return jnp.where(sel_hi, hi, lo)


def _dense_body(um_ref, im_ref, u1_ref, i1_ref, w1u_ref, w1i_ref, b1_ref,
                w2_ref, b2_ref, w3_ref, b3_ref, wpg_ref, wph_ref, bp_ref,
                o_ref):
    usel = u1_ref[...] > 0.5
    isel = i1_ref[...] > 0.5
    u_mlp = _unpack_half(um_ref[...][:, :DM], usel)
    u_gmf = _unpack_half(um_ref[...][:, DM:DM + DG], usel)
    i_mlp = _unpack_half(im_ref[...][:, :DM], isel)
    i_gmf = _unpack_half(im_ref[...][:, DM:DM + DG], isel)
    h = u_mlp @ w1u_ref[...] + i_mlp @ w1i_ref[...] + b1_ref[...]
    h = jnp.maximum(h, 0.0)
    h = jnp.maximum(h @ w2_ref[...] + b2_ref[...], 0.0)
    h = jnp.maximum(h @ w3_ref[...] + b3_ref[...], 0.0)
    gmf = u_gmf * i_gmf
    z = gmf @ wpg_ref[...] + h @ wph_ref[...] + bp_ref[...]
    o_ref[...] = 1.0 / (1.0 + jnp.exp(-z))


def _tc_dense(um, im, u1, i1, w1u, w1i, b1, w2, b2, w3, b3, wpg, wph, bp):
    BLK = 2048
    row = lambda i: (i, 0)
    rep = lambda i: (0, 0)
    return pl.pallas_call(
        _dense_body,
        grid=(B // BLK,),
        in_specs=[
            pl.BlockSpec((BLK, 128), row),
            pl.BlockSpec((BLK, 128), row),
            pl.BlockSpec((BLK, 1), row),
            pl.BlockSpec((BLK, 1), row),
            pl.BlockSpec((DM, DM), rep),
            pl.BlockSpec((DM, DM), rep),
            pl.BlockSpec((1, DM), rep),
            pl.BlockSpec((DM, DM // 2), rep),
            pl.BlockSpec((1, DM // 2), rep),
            pl.BlockSpec((DM // 2, DG), rep),
            pl.BlockSpec((1, DG), rep),
            pl.BlockSpec((DG, 1), rep),
            pl.BlockSpec((DG, 1), rep),
            pl.BlockSpec((1, 1), rep),
        ],
        out_specs=pl.BlockSpec((BLK, 1), row),
        out_shape=jax.ShapeDtypeStruct((B, 1), jnp.float32),
    )(um, im, u1, i1, w1u, w1i, b1, w2, b2, w3, b3, wpg, wph, bp)


def kernel(user, item, embed_user_GMF, embed_item_GMF, embed_user_MLP,
           embed_item_MLP, W1, b1, W2, b2, W3, b3, Wp, bp):
    ui = user.astype(jnp.int32)
    ii = item.astype(jnp.int32)
    um_idx = (ui >> 1).reshape(NW, NCH, CH)
    im_idx = (ii >> 1).reshape(NW, NCH, CH)
    t_user = _tc_pack(embed_user_MLP.T, embed_user_GMF.T)
    um_g = _sc_gather(um_idx, t_user)
    t_item = _tc_pack(embed_item_MLP.T, embed_item_GMF.T)
    im_g = _sc_gather(im_idx, t_item)
    u1 = (ui & 1).astype(jnp.float32).reshape(B, 1)
    i1 = (ii & 1).astype(jnp.float32).reshape(B, 1)
    out = _tc_dense(
        um_g, im_g, u1, i1,
        W1[:DM], W1[DM:], b1.reshape(1, DM),
        W2, b2.reshape(1, DM // 2),
        W3, b3.reshape(1, DG),
        Wp[:DG], Wp[DG:], bp.reshape(1, 1),
    )
    return out.reshape(-1)


# speedup vs baseline: 1.1462x; 1.1462x over previous
import functools

import jax
import jax.numpy as jnp
from jax import lax
from jax.experimental import pallas as pl
from jax.experimental.pallas import tpu as pltpu
from jax.experimental.pallas import tpu_sc as plsc

B = 16384
U = 1000000
DG = 16
DM = 64

_info = plsc.get_sparse_core_info()
NC = _info.num_cores
NS = _info.num_subcores
NW = NC * NS
RPW = B // NW
CH = 128
NCH = RPW // CH

UB = 8192
NUB = -(-U // UB)


def _pair_pack(x_ref):
    xb = x_ref[...].astype(jnp.bfloat16).T
    return pltpu.bitcast(xb, jnp.float32)


def _pk_body(tu_ref, ti_ref, gu_ref, gi_ref, ou_ref, oi_ref):
    pad = jnp.zeros((UB // 2, 128 - DM - DG), jnp.float32)
    ou_ref[...] = jnp.concatenate(
        [_pair_pack(tu_ref), _pair_pack(gu_ref), pad], axis=1)
    oi_ref[...] = jnp.concatenate(
        [_pair_pack(ti_ref), _pair_pack(gi_ref), pad], axis=1)


def _tc_pack(tuT, tiT, guT, giT):
    col = lambda i: (0, i)
    row = lambda i: (i, 0)
    return pl.pallas_call(
        _pk_body,
        grid=(NUB,),
        in_specs=[
            pl.BlockSpec((DM, UB), col),
            pl.BlockSpec((DM, UB), col),
            pl.BlockSpec((DG, UB), col),
            pl.BlockSpec((DG, UB), col),
        ],
        out_specs=[
            pl.BlockSpec((UB // 2, 128), row),
            pl.BlockSpec((UB // 2, 128), row),
        ],
        out_shape=[
            jax.ShapeDtypeStruct((U // 2, 128), jnp.float32),
            jax.ShapeDtypeStruct((U // 2, 128), jnp.float32),
        ],
    )(tuT, tiT, guT, giT)


@functools.partial(
    pl.kernel,
    out_type=(
        jax.ShapeDtypeStruct((B, 128), jnp.float32),
        jax.ShapeDtypeStruct((B, 128), jnp.float32),
    ),
    mesh=plsc.VectorSubcoreMesh(core_axis_name="c", subcore_axis_name="s"),
    scratch_types=[
        pltpu.VMEM((NCH, CH), jnp.int32),
        pltpu.VMEM((NCH, CH), jnp.int32),
        pltpu.VMEM((CH, 128), jnp.float32),
        pltpu.VMEM((CH, 128), jnp.float32),
        pltpu.SemaphoreType.DMA,
        pltpu.SemaphoreType.DMA,
    ],
)
def _sc_gather(uidx_hbm, iidx_hbm, tu_hbm, ti_hbm,
               um_out, im_out,
               uidx, iidx, b0, b1, s0, s1):
    wid = lax.axis_index("s") * NC + lax.axis_index("c")
    base = wid * RPW
    pltpu.sync_copy(uidx_hbm.at[wid], uidx)
    pltpu.sync_copy(iidx_hbm.at[wid], iidx)
    for j in range(NCH):
        cps = [
            pltpu.async_copy(tu_hbm.at[uidx.at[j]], b0, s0),
            pltpu.async_copy(ti_hbm.at[iidx.at[j]], b1, s1),
        ]
        for cp in cps:
            cp.wait()
        sl = pl.ds(base + j * CH, CH)
        pltpu.sync_copy(b0, um_out.at[sl])
        pltpu.sync_copy(b1, im_out.at[sl])


def _unpack_half(x, sel_hi):
    xi = jax.lax.bitcast_convert_type(x, jnp.int32)
    hi = jax.lax.bitcast_convert_type((xi >> 16) << 16, jnp.float32)
    lo = jax.lax.bitcast_convert_type(xi << 16, jnp.float32)
    return jnp.where(sel_hi, hi, lo)


def _dense_body(um_ref, im_ref, u1_ref, i1_ref, w1u_ref, w1i_ref, b1_ref,
                w2_ref, b2_ref, w3_ref, b3_ref, wpg_ref, wph_ref, bp_ref,
                o_ref):
    usel = u1_ref[...] > 0.5
    isel = i1_ref[...] > 0.5
    u_mlp = _unpack_half(um_ref[...][:, :DM], usel)
    u_gmf = _unpack_half(um_ref[...][:, DM:DM + DG], usel)
    i_mlp = _unpack_half(im_ref[...][:, :DM], isel)
    i_gmf = _unpack_half(im_ref[...][:, DM:DM + DG], isel)
    h = u_mlp @ w1u_ref[...] + i_mlp @ w1i_ref[...] + b1_ref[...]
    h = jnp.maximum(h, 0.0)
    h = jnp.maximum(h @ w2_ref[...] + b2_ref[...], 0.0)
    h = jnp.maximum(h @ w3_ref[...] + b3_ref[...], 0.0)
    gmf = u_gmf * i_gmf
    z = gmf @ wpg_ref[...] + h @ wph_ref[...] + bp_ref[...]
    o_ref[...] = 1.0 / (1.0 + jnp.exp(-z))


def _tc_dense(um, im, u1, i1, w1u, w1i, b1, w2, b2, w3, b3, wpg, wph, bp):
    BLK = 2048
    row = lambda i: (i, 0)
    rep = lambda i: (0, 0)
    return pl.pallas_call(
        _dense_body,
        grid=(B // BLK,),
        in_specs=[
            pl.BlockSpec((BLK, 128), row),
            pl.BlockSpec((BLK, 128), row),
            pl.BlockSpec((BLK, 1), row),
            pl.BlockSpec((BLK, 1), row),
            pl.BlockSpec((DM, DM), rep),
            pl.BlockSpec((DM, DM), rep),
            pl.BlockSpec((1, DM), rep),
            pl.BlockSpec((DM, DM // 2), rep),
            pl.BlockSpec((1, DM // 2), rep),
            pl.BlockSpec((DM // 2, DG), rep),
            pl.BlockSpec((1, DG), rep),
            pl.BlockSpec((DG, 1), rep),
            pl.BlockSpec((DG, 1), rep),
            pl.BlockSpec((1, 1), rep),
        ],
        out_specs=pl.BlockSpec((BLK, 1), row),
        out_shape=jax.ShapeDtypeStruct((B, 1), jnp.float32),
    )(um, im, u1, i1, w1u, w1i, b1, w2, b2, w3, b3, wpg, wph, bp)


def kernel(user, item, embed_user_GMF, embed_item_GMF, embed_user_MLP,
           embed_item_MLP, W1, b1, W2, b2, W3, b3, Wp, bp):
    t_user, t_item = _tc_pack(
        embed_user_MLP.T, embed_item_MLP.T,
        embed_user_GMF.T, embed_item_GMF.T)
    ui = user.astype(jnp.int32)
    ii = item.astype(jnp.int32)
    um_idx = (ui >> 1).reshape(NW, NCH, CH)
    im_idx = (ii >> 1).reshape(NW, NCH, CH)
    um_g, im_g = _sc_gather(um_idx, im_idx, t_user, t_item)
    u1 = (ui & 1).astype(jnp.float32).reshape(B, 1)
    i1 = (ii & 1).astype(jnp.float32).reshape(B, 1)
    out = _tc_dense(
        um_g, im_g, u1, i1,
        W1[:DM], W1[DM:], b1.reshape(1, DM),
        W2, b2.reshape(1, DM // 2),
        W3, b3.reshape(1, DG),
        Wp[:DG], Wp[DG:], bp.reshape(1, 1),
    )
    return out.reshape(-1)


# speedup vs baseline: 1.2256x; 1.0693x over previous
import functools

import jax
import jax.numpy as jnp
from jax import lax
from jax.experimental import pallas as pl
from jax.experimental.pallas import tpu as pltpu
from jax.experimental.pallas import tpu_sc as plsc

B = 16384
U = 1000000
DG = 16
DM = 64

_info = plsc.get_sparse_core_info()
NC = _info.num_cores
NS = _info.num_subcores
NW = NC * NS
RPW = B // NW
CH = 128
NCH = RPW // CH

UB = 16384
NUB = -(-U // UB)


def _pair_pack(x_ref):
    xb = x_ref[...].astype(jnp.bfloat16).T
    return pltpu.bitcast(xb, jnp.float32)


def _pk_body(tu_ref, ti_ref, gu_ref, gi_ref, ou_ref, oi_ref):
    pad = jnp.zeros((UB // 2, 128 - DM - DG), jnp.float32)
    ou_ref[...] = jnp.concatenate(
        [_pair_pack(tu_ref), _pair_pack(gu_ref), pad], axis=1)
    oi_ref[...] = jnp.concatenate(
        [_pair_pack(ti_ref), _pair_pack(gi_ref), pad], axis=1)


def _tc_pack(tuT, tiT, guT, giT):
    col = lambda i: (0, i)
    row = lambda i: (i, 0)
    return pl.pallas_call(
        _pk_body,
        grid=(NUB,),
        in_specs=[
            pl.BlockSpec((DM, UB), col),
            pl.BlockSpec((DM, UB), col),
            pl.BlockSpec((DG, UB), col),
            pl.BlockSpec((DG, UB), col),
        ],
        out_specs=[
            pl.BlockSpec((UB // 2, 128), row),
            pl.BlockSpec((UB // 2, 128), row),
        ],
        out_shape=[
            jax.ShapeDtypeStruct((U // 2, 128), jnp.float32),
            jax.ShapeDtypeStruct((U // 2, 128), jnp.float32),
        ],
    )(tuT, tiT, guT, giT)


@functools.partial(
    pl.kernel,
    out_type=(
        jax.ShapeDtypeStruct((B, 128), jnp.float32),
        jax.ShapeDtypeStruct((B, 128), jnp.float32),
    ),
    mesh=plsc.VectorSubcoreMesh(core_axis_name="c", subcore_axis_name="s"),
    scratch_types=[
        pltpu.VMEM((NCH, CH), jnp.int32),
        pltpu.VMEM((NCH, CH), jnp.int32),
        pltpu.VMEM((CH, 128), jnp.float32),
        pltpu.VMEM((CH, 128), jnp.float32),
        pltpu.SemaphoreType.DMA,
        pltpu.SemaphoreType.DMA,
    ],
)
def _sc_gather(uidx_hbm, iidx_hbm, tu_hbm, ti_hbm,
               um_out, im_out,
               uidx, iidx, b0, b1, s0, s1):
    wid = lax.axis_index("s") * NC + lax.axis_index("c")
    base = wid * RPW
    pltpu.sync_copy(uidx_hbm.at[wid], uidx)
    pltpu.sync_copy(iidx_hbm.at[wid], iidx)
    for j in range(NCH):
        cps = [
            pltpu.async_copy(tu_hbm.at[uidx.at[j]], b0, s0),
            pltpu.async_copy(ti_hbm.at[iidx.at[j]], b1, s1),
        ]
        for cp in cps:
            cp.wait()
        sl = pl.ds(base + j * CH, CH)
        pltpu.sync_copy(b0, um_out.at[sl])
        pltpu.sync_copy(b1, im_out.at[sl])


def _unpack_half(x, sel_hi):
    xi = jax.lax.bitcast_convert_type(x, jnp.int32)
    hi = jax.lax.bitcast_convert_type((xi >> 16) << 16, jnp.float32)
    lo = jax.lax.bitcast_convert_type(xi << 16, jnp.float32)
    return jnp.where(sel_hi, hi, lo)


def _dense_body(um_ref, im_ref, u1_ref, i1_ref, w1u_ref, w1i_ref, b1_ref,
                w2_ref, b2_ref, w3_ref, b3_ref, wpg_ref, wph_ref, bp_ref,
                o_ref):
    usel = u1_ref[...] > 0.5
    isel = i1_ref[...] > 0.5
    u_mlp = _unpack_half(um_ref[...][:, :DM], usel)
    u_gmf = _unpack_half(um_ref[...][:, DM:DM + DG], usel)
    i_mlp = _unpack_half(im_ref[...][:, :DM], isel)
    i_gmf = _unpack_half(im_ref[...][:, DM:DM + DG], isel)
    h = u_mlp @ w1u_ref[...] + i_mlp @ w1i_ref[...] + b1_ref[...]
    h = jnp.maximum(h, 0.0)
    h = jnp.maximum(h @ w2_ref[...] + b2_ref[...], 0.0)
    h = jnp.maximum(h @ w3_ref[...] + b3_ref[...], 0.0)
    gmf = u_gmf * i_gmf
    z = gmf @ wpg_ref[...] + h @ wph_ref[...] + bp_ref[...]
    o_ref[...] = 1.0 / (1.0 + jnp.exp(-z))


def _tc_dense(um, im, u1, i1, w1u, w1i, b1, w2, b2, w3, b3, wpg, wph, bp):
    BLK = 2048
    row = lambda i: (i, 0)
    rep = lambda i: (0, 0)
    return pl.pallas_call(
        _dense_body,
        grid=(B // BLK,),
        in_specs=[
            pl.BlockSpec((BLK, 128), row),
            pl.BlockSpec((BLK, 128), row),
            pl.BlockSpec((BLK, 1), row),
            pl.BlockSpec((BLK, 1), row),
            pl.BlockSpec((DM, DM), rep),
            pl.BlockSpec((DM, DM), rep),
            pl.BlockSpec((1, DM), rep),
            pl.BlockSpec((DM, DM // 2), rep),
            pl.BlockSpec((1, DM // 2), rep),
            pl.BlockSpec((DM // 2, DG), rep),
            pl.BlockSpec((1, DG), rep),
            pl.BlockSpec((DG, 1), rep),
            pl.BlockSpec((DG, 1), rep),
            pl.BlockSpec((1, 1), rep),
        ],
        out_specs=pl.BlockSpec((BLK, 1), row),
        out_shape=jax.ShapeDtypeStruct((B, 1), jnp.float32),
    )(um, im, u1, i1, w1u, w1i, b1, w2, b2, w3, b3, wpg, wph, bp)


def kernel(user, item, embed_user_GMF, embed_item_GMF, embed_user_MLP,
           embed_item_MLP, W1, b1, W2, b2, W3, b3, Wp, bp):
    t_user, t_item = _tc_pack(
        embed_user_MLP.T, embed_item_MLP.T,
        embed_user_GMF.T, embed_item_GMF.T)
    ui = user.astype(jnp.int32)
    ii = item.astype(jnp.int32)
    um_idx = (ui >> 1).reshape(NW, NCH, CH)
    im_idx = (ii >> 1).reshape(NW, NCH, CH)
    um_g, im_g = _sc_gather(um_idx, im_idx, t_user, t_item)
    u1 = (ui & 1).astype(jnp.float32).reshape(B, 1)
    i1 = (ii & 1).astype(jnp.float32).reshape(B, 1)
    out = _tc_dense(
        um_g, im_g, u1, i1,
        W1[:DM], W1[DM:], b1.reshape(1, DM),
        W2, b2.reshape(1, DM // 2),
        W3, b3.reshape(1, DG),
        Wp[:DG], Wp[DG:], bp.reshape(1, 1),
    )
    return out.reshape(-1)
